# Initial kernel scaffold; baseline (speedup 1.0000x reference)
import jax
import jax.numpy as jnp
from jax.experimental import pallas as pl


def kernel(h, x, edge_attr, edge_index, invar_ligand_shape, ligand_shape_emb, topo_out, e_w, params):
    raise NotImplementedError("write your pallas kernel here")


# speedup vs baseline: 9.3101x; 9.3101x over previous
import functools

import numpy as np
import jax
import jax.numpy as jnp
from jax.experimental import pallas as pl
from jax.experimental.pallas import tpu as pltpu

N = 10000
E = 160000
HID = 128
HEADS = 16
DH = HID // HEADS
SHAPE_DIM = 16
EDGE_DIM = 4
NG = 20
R_MIN, R_MAX = 0.0, 10.0
RSQRT_DH = float(1.0 / np.sqrt(DH))
STEP = (R_MAX - R_MIN) / (NG - 1)
COEFF = -0.5 / STEP**2

NODE_BLK = 1000
EDGE_BLK = 1280

_INTERPRET = False

TD_W = 400
TS_W = 272


def _ln(hdn, g, be):
    mu = jnp.mean(hdn, axis=-1, keepdims=True)
    var = jnp.mean((hdn - mu) ** 2, axis=-1, keepdims=True)
    return (hdn - mu) / jnp.sqrt(var + 1e-5) * g + be


def _full_spec(a):
    nd = a.ndim
    return pl.BlockSpec(a.shape, lambda i, *, _nd=nd: (0,) * _nd)


def _head_sum_matrix():
    j = jax.lax.broadcasted_iota(jnp.int32, (HID, HEADS), 0)
    h = jax.lax.broadcasted_iota(jnp.int32, (HID, HEADS), 1)
    return (j // DH == h).astype(jnp.float32)


def _tables_math(h, inv, xpad, wk, wv, wq):
    A_k = h @ wk[0] + inv @ wk[1] + wk[2]
    A_v = h @ wv[0] + inv @ wv[1] + wv[2]
    B_k = h @ wk[3]
    B_v = h @ wv[3]
    hdnq = h @ wq[0] + wq[1]
    q = jnp.maximum(_ln(hdnq, wq[2], wq[3]), 0.0) @ wq[4] + wq[5]
    td = jnp.concatenate([A_k, A_v, q, xpad], axis=1)
    ts = jnp.concatenate([B_k, B_v, xpad], axis=1)
    return td, ts


def _node1_body(h_ref, inv_ref, xpad_ref, *rest):
    (wk0, wk1, wk2, wk3, wv0, wv1, wv2, wv3,
     q0, q1, q2, q3, q4, q5, td_ref, ts_ref) = rest
    td, ts = _tables_math(
        h_ref[...], inv_ref[...], xpad_ref[...],
        (wk0[...], wk1[...], wk2[...], wk3[...]),
        (wv0[...], wv1[...], wv2[...], wv3[...]),
        (q0[...], q1[...], q2[...], q3[...], q4[...], q5[...]))
    td_ref[...] = td
    ts_ref[...] = ts


def _node_tables(h, inv, xpad, wk, wv, wq):
    args = [h, inv, xpad, *wk, *wv, *wq]
    in_specs = [pl.BlockSpec((NODE_BLK, h.shape[1]), lambda i: (i, 0)),
                pl.BlockSpec((NODE_BLK, SHAPE_DIM), lambda i: (i, 0)),
                pl.BlockSpec((NODE_BLK, 16), lambda i: (i, 0))]
    in_specs += [_full_spec(a) for a in args[3:]]
    return pl.pallas_call(
        _node1_body,
        grid=(N // NODE_BLK,),
        in_specs=in_specs,
        out_specs=[pl.BlockSpec((NODE_BLK, TD_W), lambda i: (i, 0)),
                   pl.BlockSpec((NODE_BLK, TS_W), lambda i: (i, 0))],
        out_shape=[jax.ShapeDtypeStruct((N, TD_W), jnp.float32),
                   jax.ShapeDtypeStruct((N, TS_W), jnp.float32)],
        interpret=_INTERPRET,
    )(*args)


def _edge_feats(gd, gs, ea):
    xd = gd[:, 384:387]
    xs = gs[:, 256:259]
    rel = xd - xs
    dist = jnp.sqrt(jnp.sum(rel * rel, axis=-1, keepdims=True) + 1e-12)
    offs = jax.lax.broadcasted_iota(jnp.int32, (1, NG), 1).astype(
        jnp.float32) * STEP
    df = jnp.exp(COEFF * (dist - offs) ** 2)
    ef = jnp.concatenate([ea] + [ea[:, a:a + 1] * df for a in range(EDGE_DIM)],
                         axis=1)
    return ef, rel


def _edge_mlp(ef, gd, gs, off, w1e, g, be, w2, b2):
    hdn = ef @ w1e + gd[:, off:off + HID] + gs[:, off:off + HID]
    return jnp.maximum(_ln(hdn, g, be), 0.0) @ w2 + b2


def _edge1_body(gd_ref, gs_ref, ea_ref, ew_ref, *rest):
    (k_w1e, k_g, k_be, k_w2, k_b2,
     v_w1e, v_g, v_be, v_w2p, v_b2p, s1_ref) = rest
    gd = gd_ref[...]
    gs = gs_ref[...]
    ef, _ = _edge_feats(gd, gs, ea_ref[...])
    kk = _edge_mlp(ef, gd, gs, 0, k_w1e[...], k_g[...], k_be[...],
                   k_w2[...], k_b2[...])
    q = gd[:, 256:384]
    logits = ((q * kk) @ _head_sum_matrix()) * RSQRT_DH
    ex = jnp.exp(logits)
    vt = _edge_mlp(ef, gd, gs, HID, v_w1e[...], v_g[...], v_be[...],
                   v_w2p[...], v_b2p[...])
    exw = ex * ew_ref[...]
    ext = jnp.concatenate([exw] * DH, axis=1)
    s1_ref[...] = jnp.concatenate([ext * vt, ex], axis=1)


def _edge1(gd, gs, ea, ew, wk, wv):
    args = [gd, gs, ea, ew, *wk, *wv]
    in_specs = [pl.BlockSpec((EDGE_BLK, TD_W), lambda i: (i, 0)),
                pl.BlockSpec((EDGE_BLK, TS_W), lambda i: (i, 0)),
                pl.BlockSpec((EDGE_BLK, EDGE_DIM), lambda i: (i, 0)),
                pl.BlockSpec((EDGE_BLK, 1), lambda i: (i, 0))]
    in_specs += [_full_spec(a) for a in args[4:]]
    return pl.pallas_call(
        _edge1_body,
        grid=(E // EDGE_BLK,),
        in_specs=in_specs,
        out_specs=pl.BlockSpec((EDGE_BLK, HID + HEADS), lambda i: (i, 0)),
        out_shape=jax.ShapeDtypeStruct((E, HID + HEADS), jnp.float32),
        interpret=_INTERPRET,
    )(*args)


def _edge2_body(gd_ref, gs_ref, ea_ref, ew_ref, *rest):
    (k_w1e, k_g, k_be, k_w2, k_b2,
     v_w1e, v_g, v_be, v_w2, v_b2, s2_ref) = rest
    gd = gd_ref[...]
    gs = gs_ref[...]
    ef, rel = _edge_feats(gd, gs, ea_ref[...])
    kk = _edge_mlp(ef, gd, gs, 0, k_w1e[...], k_g[...], k_be[...],
                   k_w2[...], k_b2[...])
    q = gd[:, 256:384]
    logits = ((q * kk) @ _head_sum_matrix()) * RSQRT_DH
    ex = jnp.exp(logits)
    v2 = _edge_mlp(ef, gd, gs, HID, v_w1e[...], v_g[...], v_be[...],
                   v_w2[...], v_b2[...])
    vv = ex * ew_ref[...] * v2
    w2 = jnp.concatenate([vv * rel[:, c:c + 1] for c in range(3)] + [ex],
                         axis=1)
    s2_ref[...] = w2


def _edge2(gd, gs, ea, ew, wk, wv):
    args = [gd, gs, ea, ew, *wk, *wv]
    in_specs = [pl.BlockSpec((EDGE_BLK, TD_W), lambda i: (i, 0)),
                pl.BlockSpec((EDGE_BLK, TS_W), lambda i: (i, 0)),
                pl.BlockSpec((EDGE_BLK, EDGE_DIM), lambda i: (i, 0)),
                pl.BlockSpec((EDGE_BLK, 1), lambda i: (i, 0))]
    in_specs += [_full_spec(a) for a in args[4:]]
    return pl.pallas_call(
        _edge2_body,
        grid=(E // EDGE_BLK,),
        in_specs=in_specs,
        out_specs=pl.BlockSpec((EDGE_BLK, 64), lambda i: (i, 0)),
        out_shape=jax.ShapeDtypeStruct((E, 64), jnp.float32),
        interpret=_INTERPRET,
    )(*args)


def _node2_body(acc_ref, h_ref, inv_ref, xpad_ref, *rest):
    (n_w1, n_b1, n_g, n_be, n_w2, n_b2,
     wk0, wk1, wk2, wk3, wv0, wv1, wv2, wv3,
     q0, q1, q2, q3, q4, q5, ho_ref, td_ref, ts_ref) = rest
    acc = acc_ref[...]
    h = h_ref[...]
    num = acc[0, :, 0:HID] + acc[1, :, 0:HID]
    den = acc[0, :, HID:HID + HEADS] + acc[1, :, HID:HID + HEADS]
    dent = jnp.concatenate([den] * DH, axis=1)
    out_t = num / (dent + 1e-16)
    u = jnp.concatenate([out_t, h], axis=1)
    hdn = u @ n_w1[...] + n_b1[...]
    ho = jnp.maximum(_ln(hdn, n_g[...], n_be[...]), 0.0) @ n_w2[...] \
        + n_b2[...] + h
    ho_ref[...] = ho
    td, ts = _tables_math(
        ho, inv_ref[...], xpad_ref[...],
        (wk0[...], wk1[...], wk2[...], wk3[...]),
        (wv0[...], wv1[...], wv2[...], wv3[...]),
        (q0[...], q1[...], q2[...], q3[...], q4[...], q5[...]))
    td_ref[...] = td
    ts_ref[...] = ts


def _node2(acc, h, inv, xpad, wn, wk, wv, wq):
    args = [acc, h, inv, xpad, *wn, *wk, *wv, *wq]
    in_specs = [pl.BlockSpec((2, NODE_BLK, HID + HEADS), lambda i: (0, i, 0)),
                pl.BlockSpec((NODE_BLK, HID), lambda i: (i, 0)),
                pl.BlockSpec((NODE_BLK, SHAPE_DIM), lambda i: (i, 0)),
                pl.BlockSpec((NODE_BLK, 16), lambda i: (i, 0))]
    in_specs += [_full_spec(a) for a in args[4:]]
    return pl.pallas_call(
        _node2_body,
        grid=(N // NODE_BLK,),
        in_specs=in_specs,
        out_specs=[pl.BlockSpec((NODE_BLK, HID), lambda i: (i, 0)),
                   pl.BlockSpec((NODE_BLK, TD_W), lambda i: (i, 0)),
                   pl.BlockSpec((NODE_BLK, TS_W), lambda i: (i, 0))],
        out_shape=[jax.ShapeDtypeStruct((N, HID), jnp.float32),
                   jax.ShapeDtypeStruct((N, TD_W), jnp.float32),
                   jax.ShapeDtypeStruct((N, TS_W), jnp.float32)],
        interpret=_INTERPRET,
    )(*args)


def _tail_body(acc_ref, x_ref, se0_ref, se1_ref, se2_ref, wft_ref, wdt_ref,
               xo_ref):
    acc = acc_ref[...]
    x = x_ref[...]
    se = (se0_ref[...], se1_ref[...], se2_ref[...])
    den = acc[0, :, 48:64] + acc[1, :, 48:64]
    wft = wft_ref[...]
    wdt = wdt_ref[...]
    outs, Ps, Ds = [], [], []
    for c in range(3):
        num = acc[0, :, c * 16:(c + 1) * 16] + acc[1, :, c * 16:(c + 1) * 16]
        oc = num / (den + 1e-16)
        outs.append(oc)
        tmp = jnp.concatenate([x[:, c:c + 1], oc, se[c]], axis=1)
        Ps.append(tmp @ wft)
        Ds.append(tmp @ wdt)
    dot = Ps[0] * Ds[0] + Ps[1] * Ds[1] + Ps[2] * Ds[2]
    dsq = Ds[0] * Ds[0] + Ds[1] * Ds[1] + Ds[2] * Ds[2]
    coef = dot / (dsq + 1e-6)
    mask = dot >= 0.0
    deltas = []
    for c in range(3):
        neg = jnp.where(mask, Ps[c], Ps[c] - coef * Ds[c])
        res = 0.2 * Ps[c] + 0.8 * neg
        delta = jnp.mean(outs[c], axis=-1, keepdims=True) \
            + jnp.mean(res, axis=-1, keepdims=True)
        deltas.append(x[:, c:c + 1] + delta)
    xo_ref[...] = jnp.concatenate(deltas, axis=1)


def _tail(acc2, x, se0, se1, se2, wft, wdt):
    args = [acc2, x, se0, se1, se2, wft, wdt]
    in_specs = [pl.BlockSpec((2, NODE_BLK, 64), lambda i: (0, i, 0)),
                pl.BlockSpec((NODE_BLK, 3), lambda i: (i, 0)),
                pl.BlockSpec((NODE_BLK, 16), lambda i: (i, 0)),
                pl.BlockSpec((NODE_BLK, 16), lambda i: (i, 0)),
                pl.BlockSpec((NODE_BLK, 16), lambda i: (i, 0)),
                _full_spec(wft), _full_spec(wdt)]
    return pl.pallas_call(
        _tail_body,
        grid=(N // NODE_BLK,),
        in_specs=in_specs,
        out_specs=pl.BlockSpec((NODE_BLK, 3), lambda i: (i, 0)),
        out_shape=jax.ShapeDtypeStruct((N, 3), jnp.float32),
        interpret=_INTERPRET,
    )(*args)


def _gather(td, ts, dst, src):
    return jnp.take(td, dst, axis=0), jnp.take(ts, src, axis=0)


def _scatter_add(rows, dst, width):
    seg = jax.ops.segment_sum(rows, dst, num_segments=N)
    return jnp.stack([seg, jnp.zeros_like(seg)])


def _prep_kv_mlp(p):
    w1 = p["W1"]
    return {
        "edge": w1[0:EDGE_DIM + NG * EDGE_DIM],
        "hd": w1[84:84 + HID],
        "hs": w1[84 + HID:84 + 2 * HID],
        "iv": w1[84 + 2 * HID:],
        "b1": p["b1"].reshape(1, -1),
        "g": p["g"].reshape(1, -1),
        "be": p["be"].reshape(1, -1),
        "W2": p["W2"],
        "b2": p["b2"].reshape(1, -1),
    }


def _prep_q_mlp(p):
    return (p["W1"], p["b1"].reshape(1, -1), p["g"].reshape(1, -1),
            p["be"].reshape(1, -1), p["W2"], p["b2"].reshape(1, -1))


def kernel(h, x, edge_attr, edge_index, invar_ligand_shape, ligand_shape_emb,
           topo_out, e_w, params):
    del topo_out
    src = edge_index[0]
    dst = edge_index[1]
    ew = e_w.reshape(E, 1)
    xpad = jnp.pad(x, ((0, 0), (0, 13)))

    perm = np.array([(j % HEADS) * DH + j // HEADS for j in range(HID)],
                    dtype=np.int32)

    px = params["x2h"]
    hk = _prep_kv_mlp(px["hk"])
    hv = _prep_kv_mlp(px["hv"])
    hq = _prep_q_mlp(px["hq"])
    no = px["node_out"]
    n_w1 = jnp.concatenate([no["W1"][0:HID][perm], no["W1"][HID:]], axis=0)
    wn = (n_w1, no["b1"].reshape(1, -1), no["g"].reshape(1, -1),
          no["be"].reshape(1, -1), no["W2"], no["b2"].reshape(1, -1))

    ph = params["h2x"]
    xk = _prep_kv_mlp(ph["xk"])
    xv = _prep_kv_mlp(ph["xv"])
    xq = _prep_q_mlp(ph["xq"])
    wft = ph["Wf"].T
    wdt = ph["Wd"].T

    def kv_pack(m):
        return (m["hd"], m["iv"], m["b1"], m["hs"])

    td1, ts1 = _node_tables(h, invar_ligand_shape, xpad,
                            kv_pack(hk), kv_pack(hv), hq)
    gd1, gs1 = _gather(td1, ts1, dst, src)
    hv_w2p = hv["W2"][:, perm]
    hv_b2p = hv["b2"][:, perm]
    s1 = _edge1(gd1, gs1, edge_attr, ew,
                (hk["edge"], hk["g"], hk["be"], hk["W2"], hk["b2"]),
                (hv["edge"], hv["g"], hv["be"], hv_w2p, hv_b2p))
    acc1 = _scatter_add(s1, dst, HID + HEADS)

    h_out, td2, ts2 = _node2(acc1, h, invar_ligand_shape, xpad, wn,
                             kv_pack(xk), kv_pack(xv), xq)

    gd2, gs2 = _gather(td2, ts2, dst, src)
    s2 = _edge2(gd2, gs2, edge_attr, ew,
                (xk["edge"], xk["g"], xk["be"], xk["W2"], xk["b2"]),
                (xv["edge"], xv["g"], xv["be"], xv["W2"],
                 xv["b2"]))
    acc2 = _scatter_add(s2, dst, 64)

    se0 = ligand_shape_emb[:, :, 0]
    se1 = ligand_shape_emb[:, :, 1]
    se2 = ligand_shape_emb[:, :, 2]
    x_out = _tail(acc2, x, se0, se1, se2, wft, wdt)
    return h_out, x_out


# speedup vs baseline: 18.0282x; 1.9364x over previous
import functools

import numpy as np
import jax
import jax.numpy as jnp
from jax import lax
from jax.experimental import pallas as pl
from jax.experimental.pallas import tpu as pltpu
from jax.experimental.pallas import tpu_sc as plsc

N = 10000
E = 160000
HID = 128
HEADS = 16
DH = HID // HEADS
SHAPE_DIM = 16
EDGE_DIM = 4
NG = 20
R_MIN, R_MAX = 0.0, 10.0
RSQRT_DH = float(1.0 / np.sqrt(DH))
STEP = (R_MAX - R_MIN) / (NG - 1)
COEFF = -0.5 / STEP**2

NODE_BLK = 1000
EDGE_BLK = 1280

_INTERPRET = False

TD_W = 384
TS_W = 256


def _ln(hdn, g, be):
    mu = jnp.mean(hdn, axis=-1, keepdims=True)
    var = jnp.mean((hdn - mu) ** 2, axis=-1, keepdims=True)
    return (hdn - mu) / jnp.sqrt(var + 1e-5) * g + be


def _full_spec(a):
    nd = a.ndim
    return pl.BlockSpec(a.shape, lambda i, *, _nd=nd: (0,) * _nd)


def _head_sum_matrix():
    j = jax.lax.broadcasted_iota(jnp.int32, (HID, HEADS), 0)
    h = jax.lax.broadcasted_iota(jnp.int32, (HID, HEADS), 1)
    return (j // DH == h).astype(jnp.float32)


def _tables_math(h, inv, wk, wv, wq):
    A_k = h @ wk[0] + inv @ wk[1] + wk[2]
    A_v = h @ wv[0] + inv @ wv[1] + wv[2]
    B_k = h @ wk[3]
    B_v = h @ wv[3]
    hdnq = h @ wq[0] + wq[1]
    q = jnp.maximum(_ln(hdnq, wq[2], wq[3]), 0.0) @ wq[4] + wq[5]
    td = jnp.concatenate([A_k, A_v, q], axis=1)
    ts = jnp.concatenate([B_k, B_v], axis=1)
    return td, ts


def _node1_body(h_ref, inv_ref, *rest):
    (wk0, wk1, wk2, wk3, wv0, wv1, wv2, wv3,
     q0, q1, q2, q3, q4, q5, td_ref, ts_ref) = rest
    td, ts = _tables_math(
        h_ref[...], inv_ref[...],
        (wk0[...], wk1[...], wk2[...], wk3[...]),
        (wv0[...], wv1[...], wv2[...], wv3[...]),
        (q0[...], q1[...], q2[...], q3[...], q4[...], q5[...]))
    td_ref[...] = td
    ts_ref[...] = ts


def _node_tables(h, inv, wk, wv, wq):
    args = [h, inv, *wk, *wv, *wq]
    in_specs = [pl.BlockSpec((NODE_BLK, h.shape[1]), lambda i: (i, 0)),
                pl.BlockSpec((NODE_BLK, SHAPE_DIM), lambda i: (i, 0))]
    in_specs += [_full_spec(a) for a in args[2:]]
    return pl.pallas_call(
        _node1_body,
        grid=(N // NODE_BLK,),
        in_specs=in_specs,
        out_specs=[pl.BlockSpec((NODE_BLK, TD_W), lambda i: (i, 0)),
                   pl.BlockSpec((NODE_BLK, TS_W), lambda i: (i, 0))],
        out_shape=[jax.ShapeDtypeStruct((N, TD_W), jnp.float32),
                   jax.ShapeDtypeStruct((N, TS_W), jnp.float32)],
        interpret=_INTERPRET,
    )(*args)


def _edge_feats(gxd, gxs, ea):
    rel = gxd[:, 0:3] - gxs[:, 0:3]
    dist = jnp.sqrt(jnp.sum(rel * rel, axis=-1, keepdims=True) + 1e-12)
    offs = jax.lax.broadcasted_iota(jnp.int32, (1, NG), 1).astype(
        jnp.float32) * STEP
    df = jnp.exp(COEFF * (dist - offs) ** 2)
    ef = jnp.concatenate([ea] + [ea[:, a:a + 1] * df for a in range(EDGE_DIM)],
                         axis=1)
    return ef, rel


def _edge_mlp(ef, gd, gs, off, w1e, g, be, w2, b2):
    hdn = ef @ w1e + gd[:, off:off + HID] + gs[:, off:off + HID]
    return jnp.maximum(_ln(hdn, g, be), 0.0) @ w2 + b2


def _edge1_body(gd_ref, gs_ref, gxd_ref, gxs_ref, ea_ref, ew_ref, *rest):
    (k_w1e, k_g, k_be, k_w2, k_b2,
     v_w1e, v_g, v_be, v_w2p, v_b2p, s1_ref) = rest
    gd = gd_ref[...]
    gs = gs_ref[...]
    ef, _ = _edge_feats(gxd_ref[...], gxs_ref[...], ea_ref[...])
    kk = _edge_mlp(ef, gd, gs, 0, k_w1e[...], k_g[...], k_be[...],
                   k_w2[...], k_b2[...])
    q = gd[:, 256:384]
    logits = ((q * kk) @ _head_sum_matrix()) * RSQRT_DH
    ex = jnp.exp(logits)
    vt = _edge_mlp(ef, gd, gs, HID, v_w1e[...], v_g[...], v_be[...],
                   v_w2p[...], v_b2p[...])
    exw = ex * ew_ref[...]
    ext = jnp.concatenate([exw] * DH, axis=1)
    s1_ref[...] = jnp.concatenate([ext * vt, ex], axis=1)


def _edge1(gd, gs, gxd, gxs, ea, ew, wk, wv):
    args = [gd, gs, gxd, gxs, ea, ew, *wk, *wv]
    in_specs = [pl.BlockSpec((EDGE_BLK, TD_W), lambda i: (i, 0)),
                pl.BlockSpec((EDGE_BLK, TS_W), lambda i: (i, 0)),
                pl.BlockSpec((EDGE_BLK, 16), lambda i: (i, 0)),
                pl.BlockSpec((EDGE_BLK, 16), lambda i: (i, 0)),
                pl.BlockSpec((EDGE_BLK, EDGE_DIM), lambda i: (i, 0)),
                pl.BlockSpec((EDGE_BLK, 1), lambda i: (i, 0))]
    in_specs += [_full_spec(a) for a in args[6:]]
    return pl.pallas_call(
        _edge1_body,
        grid=(E // EDGE_BLK,),
        in_specs=in_specs,
        out_specs=pl.BlockSpec((EDGE_BLK, HID + HEADS), lambda i: (i, 0)),
        out_shape=jax.ShapeDtypeStruct((E, HID + HEADS), jnp.float32),
        interpret=_INTERPRET,
    )(*args)


def _edge2_body(gd_ref, gs_ref, gxd_ref, gxs_ref, ea_ref, ew_ref, *rest):
    (k_w1e, k_g, k_be, k_w2, k_b2,
     v_w1e, v_g, v_be, v_w2, v_b2, s2_ref) = rest
    gd = gd_ref[...]
    gs = gs_ref[...]
    ef, rel = _edge_feats(gxd_ref[...], gxs_ref[...], ea_ref[...])
    kk = _edge_mlp(ef, gd, gs, 0, k_w1e[...], k_g[...], k_be[...],
                   k_w2[...], k_b2[...])
    q = gd[:, 256:384]
    logits = ((q * kk) @ _head_sum_matrix()) * RSQRT_DH
    ex = jnp.exp(logits)
    v2 = _edge_mlp(ef, gd, gs, HID, v_w1e[...], v_g[...], v_be[...],
                   v_w2[...], v_b2[...])
    vv = ex * ew_ref[...] * v2
    w2 = jnp.concatenate([vv * rel[:, c:c + 1] for c in range(3)] + [ex],
                         axis=1)
    s2_ref[...] = w2


def _edge2(gd, gs, gxd, gxs, ea, ew, wk, wv):
    args = [gd, gs, gxd, gxs, ea, ew, *wk, *wv]
    in_specs = [pl.BlockSpec((EDGE_BLK, TD_W), lambda i: (i, 0)),
                pl.BlockSpec((EDGE_BLK, TS_W), lambda i: (i, 0)),
                pl.BlockSpec((EDGE_BLK, 16), lambda i: (i, 0)),
                pl.BlockSpec((EDGE_BLK, 16), lambda i: (i, 0)),
                pl.BlockSpec((EDGE_BLK, EDGE_DIM), lambda i: (i, 0)),
                pl.BlockSpec((EDGE_BLK, 1), lambda i: (i, 0))]
    in_specs += [_full_spec(a) for a in args[6:]]
    return pl.pallas_call(
        _edge2_body,
        grid=(E // EDGE_BLK,),
        in_specs=in_specs,
        out_specs=pl.BlockSpec((EDGE_BLK, 64), lambda i: (i, 0)),
        out_shape=jax.ShapeDtypeStruct((E, 64), jnp.float32),
        interpret=_INTERPRET,
    )(*args)


def _node2_body(acc_ref, h_ref, inv_ref, *rest):
    (n_w1, n_b1, n_g, n_be, n_w2, n_b2,
     wk0, wk1, wk2, wk3, wv0, wv1, wv2, wv3,
     q0, q1, q2, q3, q4, q5, ho_ref, td_ref, ts_ref) = rest
    acc = acc_ref[...]
    h = h_ref[...]
    num = acc[0, :, 0:HID] + acc[1, :, 0:HID]
    den = acc[0, :, HID:HID + HEADS] + acc[1, :, HID:HID + HEADS]
    dent = jnp.concatenate([den] * DH, axis=1)
    out_t = num / (dent + 1e-16)
    u = jnp.concatenate([out_t, h], axis=1)
    hdn = u @ n_w1[...] + n_b1[...]
    ho = jnp.maximum(_ln(hdn, n_g[...], n_be[...]), 0.0) @ n_w2[...] \
        + n_b2[...] + h
    ho_ref[...] = ho
    td, ts = _tables_math(
        ho, inv_ref[...],
        (wk0[...], wk1[...], wk2[...], wk3[...]),
        (wv0[...], wv1[...], wv2[...], wv3[...]),
        (q0[...], q1[...], q2[...], q3[...], q4[...], q5[...]))
    td_ref[...] = td
    ts_ref[...] = ts


def _node2(acc, h, inv, wn, wk, wv, wq):
    args = [acc, h, inv, *wn, *wk, *wv, *wq]
    in_specs = [pl.BlockSpec((2, NODE_BLK, HID + HEADS), lambda i: (0, i, 0)),
                pl.BlockSpec((NODE_BLK, HID), lambda i: (i, 0)),
                pl.BlockSpec((NODE_BLK, SHAPE_DIM), lambda i: (i, 0))]
    in_specs += [_full_spec(a) for a in args[3:]]
    return pl.pallas_call(
        _node2_body,
        grid=(N // NODE_BLK,),
        in_specs=in_specs,
        out_specs=[pl.BlockSpec((NODE_BLK, HID), lambda i: (i, 0)),
                   pl.BlockSpec((NODE_BLK, TD_W), lambda i: (i, 0)),
                   pl.BlockSpec((NODE_BLK, TS_W), lambda i: (i, 0))],
        out_shape=[jax.ShapeDtypeStruct((N, HID), jnp.float32),
                   jax.ShapeDtypeStruct((N, TD_W), jnp.float32),
                   jax.ShapeDtypeStruct((N, TS_W), jnp.float32)],
        interpret=_INTERPRET,
    )(*args)


def _tail_body(acc_ref, x_ref, se0_ref, se1_ref, se2_ref, wft_ref, wdt_ref,
               xo_ref):
    acc = acc_ref[...]
    x = x_ref[...]
    se = (se0_ref[...], se1_ref[...], se2_ref[...])
    den = acc[0, :, 48:64] + acc[1, :, 48:64]
    wft = wft_ref[...]
    wdt = wdt_ref[...]
    outs, Ps, Ds = [], [], []
    for c in range(3):
        num = acc[0, :, c * 16:(c + 1) * 16] + acc[1, :, c * 16:(c + 1) * 16]
        oc = num / (den + 1e-16)
        outs.append(oc)
        tmp = jnp.concatenate([x[:, c:c + 1], oc, se[c]], axis=1)
        Ps.append(tmp @ wft)
        Ds.append(tmp @ wdt)
    dot = Ps[0] * Ds[0] + Ps[1] * Ds[1] + Ps[2] * Ds[2]
    dsq = Ds[0] * Ds[0] + Ds[1] * Ds[1] + Ds[2] * Ds[2]
    coef = dot / (dsq + 1e-6)
    mask = dot >= 0.0
    deltas = []
    for c in range(3):
        neg = jnp.where(mask, Ps[c], Ps[c] - coef * Ds[c])
        res = 0.2 * Ps[c] + 0.8 * neg
        delta = jnp.mean(outs[c], axis=-1, keepdims=True) \
            + jnp.mean(res, axis=-1, keepdims=True)
        deltas.append(x[:, c:c + 1] + delta)
    xo_ref[...] = jnp.concatenate(deltas, axis=1)


def _tail(acc2, x, se0, se1, se2, wft, wdt):
    args = [acc2, x, se0, se1, se2, wft, wdt]
    in_specs = [pl.BlockSpec((2, NODE_BLK, 64), lambda i: (0, i, 0)),
                pl.BlockSpec((NODE_BLK, 3), lambda i: (i, 0)),
                pl.BlockSpec((NODE_BLK, 16), lambda i: (i, 0)),
                pl.BlockSpec((NODE_BLK, 16), lambda i: (i, 0)),
                pl.BlockSpec((NODE_BLK, 16), lambda i: (i, 0)),
                _full_spec(wft), _full_spec(wdt)]
    return pl.pallas_call(
        _tail_body,
        grid=(N // NODE_BLK,),
        in_specs=in_specs,
        out_specs=pl.BlockSpec((NODE_BLK, 3), lambda i: (i, 0)),
        out_shape=jax.ShapeDtypeStruct((N, 3), jnp.float32),
        interpret=_INTERPRET,
    )(*args)


CHUNK = 128
NROWS = E // CHUNK
NWORK = 32
ROWS_PER_W = -(-NROWS // NWORK)
NODES_PER_SUB = N // 16

_SC_MESH = plsc.VectorSubcoreMesh(core_axis_name="c", subcore_axis_name="s")


def _gather(td, ts, dst_m, src_m):
    @functools.partial(
        pl.kernel,
        out_type=[jax.ShapeDtypeStruct((E, TD_W), jnp.float32),
                  jax.ShapeDtypeStruct((E, TS_W), jnp.float32)],
        mesh=_SC_MESH,
        scratch_types=[pltpu.VMEM((CHUNK,), jnp.int32),
                       pltpu.VMEM((CHUNK,), jnp.int32),
                       pltpu.VMEM((CHUNK, TD_W), jnp.float32),
                       pltpu.VMEM((CHUNK, TS_W), jnp.float32),
                       pltpu.SemaphoreType.DMA,
                       pltpu.SemaphoreType.DMA],
    )
    def gk(td_hbm, ts_hbm, dm_hbm, sm_hbm, gd_hbm, gs_hbm,
           idx_d, idx_s, rows_d, rows_s, sem_d, sem_s):
        wid = lax.axis_index("s") * 2 + lax.axis_index("c")

        @pl.loop(0, ROWS_PER_W)
        def _(i):
            r = wid + i * NWORK

            @pl.when(r < NROWS)
            def _():
                pltpu.sync_copy(dm_hbm.at[r], idx_d)
                pltpu.sync_copy(sm_hbm.at[r], idx_s)
                cd = pltpu.async_copy(td_hbm.at[idx_d], rows_d, sem_d)
                cs = pltpu.async_copy(ts_hbm.at[idx_s], rows_s, sem_s)
                cd.wait()
                cs.wait()
                pltpu.sync_copy(rows_d, gd_hbm.at[pl.ds(r * CHUNK, CHUNK)])
                pltpu.sync_copy(rows_s, gs_hbm.at[pl.ds(r * CHUNK, CHUNK)])

    return gk(td, ts, dst_m, src_m)


def _gather_x(xpad, dst_m, src_m):
    @functools.partial(
        pl.kernel,
        out_type=[jax.ShapeDtypeStruct((E, 16), jnp.float32),
                  jax.ShapeDtypeStruct((E, 16), jnp.float32)],
        mesh=_SC_MESH,
        scratch_types=[pltpu.VMEM((CHUNK,), jnp.int32),
                       pltpu.VMEM((CHUNK,), jnp.int32),
                       pltpu.VMEM((CHUNK, 16), jnp.float32),
                       pltpu.VMEM((CHUNK, 16), jnp.float32),
                       pltpu.SemaphoreType.DMA,
                       pltpu.SemaphoreType.DMA],
        compiler_params=pltpu.CompilerParams(use_tc_tiling_on_sc=False),
    )
    def gxk(x_hbm, dm_hbm, sm_hbm, gxd_hbm, gxs_hbm,
            idx_d, idx_s, rows_d, rows_s, sem_d, sem_s):
        wid = lax.axis_index("s") * 2 + lax.axis_index("c")

        @pl.loop(0, ROWS_PER_W)
        def _(i):
            r = wid + i * NWORK

            @pl.when(r < NROWS)
            def _():
                pltpu.sync_copy(dm_hbm.at[r], idx_d)
                pltpu.sync_copy(sm_hbm.at[r], idx_s)
                cd = pltpu.async_copy(x_hbm.at[idx_d], rows_d, sem_d)
                cs = pltpu.async_copy(x_hbm.at[idx_s], rows_s, sem_s)
                cd.wait()
                cs.wait()
                pltpu.sync_copy(rows_d, gxd_hbm.at[pl.ds(r * CHUNK, CHUNK)])
                pltpu.sync_copy(rows_s, gxs_hbm.at[pl.ds(r * CHUNK, CHUNK)])

    return gxk(xpad, dst_m, src_m)


def _scatter_add(rows, dst_m, width):
    zeros = jnp.zeros((NODES_PER_SUB, width), jnp.float32)

    @functools.partial(
        pl.kernel,
        out_type=jax.ShapeDtypeStruct((2, N, width), jnp.float32),
        mesh=_SC_MESH,
        scratch_types=[pltpu.VMEM((CHUNK,), jnp.int32),
                       pltpu.VMEM((CHUNK, width), jnp.float32),
                       pltpu.VMEM_SHARED((N, width), jnp.float32)],
        compiler_params=pltpu.CompilerParams(use_tc_tiling_on_sc=False),
    )
    def sk(rows_hbm, dm_hbm, z_hbm, acc_hbm, idx, buf, shared):
        cid = lax.axis_index("c")
        sid = lax.axis_index("s")
        wid = sid * 2 + cid
        pltpu.sync_copy(z_hbm, shared.at[pl.ds(sid * NODES_PER_SUB,
                                               NODES_PER_SUB)])
        plsc.subcore_barrier()

        @pl.loop(0, ROWS_PER_W)
        def _(i):
            r = wid + i * NWORK

            @pl.when(r < NROWS)
            def _():
                pltpu.sync_copy(dm_hbm.at[r], idx)
                pltpu.sync_copy(rows_hbm.at[pl.ds(r * CHUNK, CHUNK)], buf)
                pltpu.sync_copy(buf, shared.at[idx], add=True)

        plsc.subcore_barrier()
        pltpu.sync_copy(
            shared.at[pl.ds(sid * NODES_PER_SUB, NODES_PER_SUB)],
            acc_hbm.at[cid, pl.ds(sid * NODES_PER_SUB, NODES_PER_SUB)])

    return sk(rows, dst_m, zeros)


def _prep_kv_mlp(p):
    w1 = p["W1"]
    return {
        "edge": w1[0:EDGE_DIM + NG * EDGE_DIM],
        "hd": w1[84:84 + HID],
        "hs": w1[84 + HID:84 + 2 * HID],
        "iv": w1[84 + 2 * HID:],
        "b1": p["b1"].reshape(1, -1),
        "g": p["g"].reshape(1, -1),
        "be": p["be"].reshape(1, -1),
        "W2": p["W2"],
        "b2": p["b2"].reshape(1, -1),
    }


def _prep_q_mlp(p):
    return (p["W1"], p["b1"].reshape(1, -1), p["g"].reshape(1, -1),
            p["be"].reshape(1, -1), p["W2"], p["b2"].reshape(1, -1))


def kernel(h, x, edge_attr, edge_index, invar_ligand_shape, ligand_shape_emb,
           topo_out, e_w, params):
    del topo_out
    src = edge_index[0]
    dst = edge_index[1]
    dst_m = dst.reshape(NROWS, CHUNK)
    src_m = src.reshape(NROWS, CHUNK)
    ew = e_w.reshape(E, 1)
    xpad = jnp.pad(x, ((0, 0), (0, 13)))

    perm = np.array([(j % HEADS) * DH + j // HEADS for j in range(HID)],
                    dtype=np.int32)

    px = params["x2h"]
    hk = _prep_kv_mlp(px["hk"])
    hv = _prep_kv_mlp(px["hv"])
    hq = _prep_q_mlp(px["hq"])
    no = px["node_out"]
    n_w1 = jnp.concatenate([no["W1"][0:HID][perm], no["W1"][HID:]], axis=0)
    wn = (n_w1, no["b1"].reshape(1, -1), no["g"].reshape(1, -1),
          no["be"].reshape(1, -1), no["W2"], no["b2"].reshape(1, -1))

    ph = params["h2x"]
    xk = _prep_kv_mlp(ph["xk"])
    xv = _prep_kv_mlp(ph["xv"])
    xq = _prep_q_mlp(ph["xq"])
    wft = ph["Wf"].T
    wdt = ph["Wd"].T

    def kv_pack(m):
        return (m["hd"], m["iv"], m["b1"], m["hs"])

    gxd, gxs = _gather_x(xpad, dst_m, src_m)

    td1, ts1 = _node_tables(h, invar_ligand_shape,
                            kv_pack(hk), kv_pack(hv), hq)
    gd1, gs1 = _gather(td1, ts1, dst_m, src_m)
    hv_w2p = hv["W2"][:, perm]
    hv_b2p = hv["b2"][:, perm]
    s1 = _edge1(gd1, gs1, gxd, gxs, edge_attr, ew,
                (hk["edge"], hk["g"], hk["be"], hk["W2"], hk["b2"]),
                (hv["edge"], hv["g"], hv["be"], hv_w2p, hv_b2p))
    acc1 = _scatter_add(s1, dst_m, HID + HEADS)

    h_out, td2, ts2 = _node2(acc1, h, invar_ligand_shape, wn,
                             kv_pack(xk), kv_pack(xv), xq)

    gd2, gs2 = _gather(td2, ts2, dst_m, src_m)
    s2 = _edge2(gd2, gs2, gxd, gxs, edge_attr, ew,
                (xk["edge"], xk["g"], xk["be"], xk["W2"], xk["b2"]),
                (xv["edge"], xv["g"], xv["be"], xv["W2"],
                 xv["b2"]))
    acc2 = _scatter_add(s2, dst_m, 64)

    se0 = ligand_shape_emb[:, :, 0]
    se1 = ligand_shape_emb[:, :, 1]
    se2 = ligand_shape_emb[:, :, 2]
    x_out = _tail(acc2, x, se0, se1, se2, wft, wdt)
    return h_out, x_out


# speedup vs baseline: 21.0525x; 1.1678x over previous
import functools

import numpy as np
import jax
import jax.numpy as jnp
from jax import lax
from jax.experimental import pallas as pl
from jax.experimental.pallas import tpu as pltpu
from jax.experimental.pallas import tpu_sc as plsc

N = 10000
E = 160000
HID = 128
HEADS = 16
DH = HID // HEADS
SHAPE_DIM = 16
EDGE_DIM = 4
NG = 20
R_MIN, R_MAX = 0.0, 10.0
RSQRT_DH = float(1.0 / np.sqrt(DH))
STEP = (R_MAX - R_MIN) / (NG - 1)
COEFF = -0.5 / STEP**2

NODE_BLK = 1000
EDGE_BLK = 1280

_INTERPRET = False

TD_W = 384
TS_W = 256


def _ln_mxu(hdn, g, be):
    j = jnp.full((HID, HID), 1.0 / HID, jnp.float32)
    mu = hdn @ j
    ex2 = (hdn * hdn) @ j
    var = ex2 - mu * mu
    return (hdn - mu) * jax.lax.rsqrt(var + 1e-5) * g + be


def _sel(rows, cols, fn):
    r = jax.lax.broadcasted_iota(jnp.int32, (rows, cols), 0)
    c = jax.lax.broadcasted_iota(jnp.int32, (rows, cols), 1)
    return fn(r, c).astype(jnp.float32)


def _full_spec(a):
    nd = a.ndim
    return pl.BlockSpec(a.shape, lambda i, *, _nd=nd: (0,) * _nd)


def _head_sum_matrix():
    j = jax.lax.broadcasted_iota(jnp.int32, (HID, HEADS), 0)
    h = jax.lax.broadcasted_iota(jnp.int32, (HID, HEADS), 1)
    return (j // DH == h).astype(jnp.float32)


def _tables_math(h, inv, wk, wv, wq):
    A_k = h @ wk[0] + inv @ wk[1] + wk[2]
    A_v = h @ wv[0] + inv @ wv[1] + wv[2]
    B_k = h @ wk[3]
    B_v = h @ wv[3]
    hdnq = h @ wq[0] + wq[1]
    q = jnp.maximum(_ln_mxu(hdnq, wq[2], wq[3]), 0.0) @ wq[4] + wq[5]
    td = jnp.concatenate([A_k, A_v, q], axis=1)
    ts = jnp.concatenate([B_k, B_v], axis=1)
    return td, ts


def _node1_body(h_ref, inv_ref, *rest):
    (wk0, wk1, wk2, wk3, wv0, wv1, wv2, wv3,
     q0, q1, q2, q3, q4, q5, td_ref, ts_ref) = rest
    td, ts = _tables_math(
        h_ref[...], inv_ref[...],
        (wk0[...], wk1[...], wk2[...], wk3[...]),
        (wv0[...], wv1[...], wv2[...], wv3[...]),
        (q0[...], q1[...], q2[...], q3[...], q4[...], q5[...]))
    td_ref[...] = td
    ts_ref[...] = ts


def _node_tables(h, inv, wk, wv, wq):
    args = [h, inv, *wk, *wv, *wq]
    in_specs = [pl.BlockSpec((NODE_BLK, h.shape[1]), lambda i: (i, 0)),
                pl.BlockSpec((NODE_BLK, SHAPE_DIM), lambda i: (i, 0))]
    in_specs += [_full_spec(a) for a in args[2:]]
    return pl.pallas_call(
        _node1_body,
        grid=(N // NODE_BLK,),
        in_specs=in_specs,
        out_specs=[pl.BlockSpec((NODE_BLK, TD_W), lambda i: (i, 0)),
                   pl.BlockSpec((NODE_BLK, TS_W), lambda i: (i, 0))],
        out_shape=[jax.ShapeDtypeStruct((N, TD_W), jnp.float32),
                   jax.ShapeDtypeStruct((N, TS_W), jnp.float32)],
        interpret=_INTERPRET,
    )(*args)


def _edge_feats(gxd, gxs, ea):
    rel = gxd - gxs
    d2 = (rel * rel) @ jnp.ones((16, NG), jnp.float32)
    dist = jnp.sqrt(d2 + 1e-12)
    offs = jax.lax.broadcasted_iota(jnp.int32, (1, NG), 1).astype(
        jnp.float32) * STEP
    df = jnp.exp(COEFF * (dist - offs) ** 2)
    ea_b = ea @ _sel(EDGE_DIM, NG * EDGE_DIM, lambda r, c: c // NG == r)
    df_b = df @ _sel(NG, NG * EDGE_DIM, lambda r, c: c % NG == r)
    return ea_b * df_b, rel


def _edge_mlp(ea, rf, gd, gs, off, w1a, w1r, g, be, w2, b2):
    hdn = ea @ w1a + rf @ w1r + gd[:, off:off + HID] + gs[:, off:off + HID]
    return jnp.maximum(_ln_mxu(hdn, g, be), 0.0) @ w2 + b2


def _edge1_body(gd_ref, gs_ref, gxd_ref, gxs_ref, ea_ref, ew_ref, *rest):
    (k_w1a, k_w1r, k_g, k_be, k_w2, k_b2,
     v_w1a, v_w1r, v_g, v_be, v_w2p, v_b2p, s1_ref) = rest
    gd = gd_ref[...]
    gs = gs_ref[...]
    ea = ea_ref[...]
    rf, _ = _edge_feats(gxd_ref[...], gxs_ref[...], ea)
    kk = _edge_mlp(ea, rf, gd, gs, 0, k_w1a[...], k_w1r[...], k_g[...],
                   k_be[...], k_w2[...], k_b2[...])
    q = gd[:, 256:384]
    logits = ((q * kk) @ _head_sum_matrix()) * RSQRT_DH
    ex = jnp.exp(logits)
    vt = _edge_mlp(ea, rf, gd, gs, HID, v_w1a[...], v_w1r[...], v_g[...],
                   v_be[...], v_w2p[...], v_b2p[...])
    exw = ex * (ew_ref[...] @ jnp.ones((1, HEADS), jnp.float32))
    ext = exw @ _sel(HEADS, HID, lambda r, c: c % HEADS == r)
    s1_ref[...] = jnp.concatenate([ext * vt, ex], axis=1)


def _edge1(gd, gs, gxd, gxs, ea, ew, wk, wv):
    args = [gd, gs, gxd, gxs, ea, ew, *wk, *wv]
    in_specs = [pl.BlockSpec((EDGE_BLK, TD_W), lambda i: (i, 0)),
                pl.BlockSpec((EDGE_BLK, TS_W), lambda i: (i, 0)),
                pl.BlockSpec((EDGE_BLK, 16), lambda i: (i, 0)),
                pl.BlockSpec((EDGE_BLK, 16), lambda i: (i, 0)),
                pl.BlockSpec((EDGE_BLK, EDGE_DIM), lambda i: (i, 0)),
                pl.BlockSpec((EDGE_BLK, 1), lambda i: (i, 0))]
    in_specs += [_full_spec(a) for a in args[6:]]
    return pl.pallas_call(
        _edge1_body,
        grid=(E // EDGE_BLK,),
        in_specs=in_specs,
        out_specs=pl.BlockSpec((EDGE_BLK, HID + HEADS), lambda i: (i, 0)),
        out_shape=jax.ShapeDtypeStruct((E, HID + HEADS), jnp.float32),
        interpret=_INTERPRET,
    )(*args)


def _edge2_body(gd_ref, gs_ref, gxd_ref, gxs_ref, ea_ref, ew_ref, *rest):
    (k_w1a, k_w1r, k_g, k_be, k_w2, k_b2,
     v_w1a, v_w1r, v_g, v_be, v_w2, v_b2, s2_ref) = rest
    gd = gd_ref[...]
    gs = gs_ref[...]
    ea = ea_ref[...]
    rf, rel = _edge_feats(gxd_ref[...], gxs_ref[...], ea)
    kk = _edge_mlp(ea, rf, gd, gs, 0, k_w1a[...], k_w1r[...], k_g[...],
                   k_be[...], k_w2[...], k_b2[...])
    q = gd[:, 256:384]
    logits = ((q * kk) @ _head_sum_matrix()) * RSQRT_DH
    ex = jnp.exp(logits)
    v2 = _edge_mlp(ea, rf, gd, gs, HID, v_w1a[...], v_w1r[...], v_g[...],
                   v_be[...], v_w2[...], v_b2[...])
    vv = ex * (ew_ref[...] @ jnp.ones((1, HEADS), jnp.float32)) * v2
    vv3 = vv @ _sel(HEADS, 48, lambda r, c: c % HEADS == r)
    rel3 = rel @ _sel(16, 48, lambda r, c: c // HEADS == r)
    s2_ref[...] = jnp.concatenate([vv3 * rel3, ex], axis=1)


def _edge2(gd, gs, gxd, gxs, ea, ew, wk, wv):
    args = [gd, gs, gxd, gxs, ea, ew, *wk, *wv]
    in_specs = [pl.BlockSpec((EDGE_BLK, TD_W), lambda i: (i, 0)),
                pl.BlockSpec((EDGE_BLK, TS_W), lambda i: (i, 0)),
                pl.BlockSpec((EDGE_BLK, 16), lambda i: (i, 0)),
                pl.BlockSpec((EDGE_BLK, 16), lambda i: (i, 0)),
                pl.BlockSpec((EDGE_BLK, EDGE_DIM), lambda i: (i, 0)),
                pl.BlockSpec((EDGE_BLK, 1), lambda i: (i, 0))]
    in_specs += [_full_spec(a) for a in args[6:]]
    return pl.pallas_call(
        _edge2_body,
        grid=(E // EDGE_BLK,),
        in_specs=in_specs,
        out_specs=pl.BlockSpec((EDGE_BLK, 64), lambda i: (i, 0)),
        out_shape=jax.ShapeDtypeStruct((E, 64), jnp.float32),
        interpret=_INTERPRET,
    )(*args)


def _node2_body(acc_ref, h_ref, inv_ref, *rest):
    (n_w1, n_b1, n_g, n_be, n_w2, n_b2,
     wk0, wk1, wk2, wk3, wv0, wv1, wv2, wv3,
     q0, q1, q2, q3, q4, q5, ho_ref, td_ref, ts_ref) = rest
    acc = acc_ref[...]
    h = h_ref[...]
    num = acc[0, :, 0:HID] + acc[1, :, 0:HID]
    den = acc[0, :, HID:HID + HEADS] + acc[1, :, HID:HID + HEADS]
    dent = jnp.concatenate([den] * DH, axis=1)
    out_t = num / (dent + 1e-16)
    u = jnp.concatenate([out_t, h], axis=1)
    hdn = u @ n_w1[...] + n_b1[...]
    ho = jnp.maximum(_ln_mxu(hdn, n_g[...], n_be[...]), 0.0) @ n_w2[...] \
        + n_b2[...] + h
    ho_ref[...] = ho
    td, ts = _tables_math(
        ho, inv_ref[...],
        (wk0[...], wk1[...], wk2[...], wk3[...]),
        (wv0[...], wv1[...], wv2[...], wv3[...]),
        (q0[...], q1[...], q2[...], q3[...], q4[...], q5[...]))
    td_ref[...] = td
    ts_ref[...] = ts


def _node2(acc, h, inv, wn, wk, wv, wq):
    args = [acc, h, inv, *wn, *wk, *wv, *wq]
    in_specs = [pl.BlockSpec((2, NODE_BLK, HID + HEADS), lambda i: (0, i, 0)),
                pl.BlockSpec((NODE_BLK, HID), lambda i: (i, 0)),
                pl.BlockSpec((NODE_BLK, SHAPE_DIM), lambda i: (i, 0))]
    in_specs += [_full_spec(a) for a in args[3:]]
    return pl.pallas_call(
        _node2_body,
        grid=(N // NODE_BLK,),
        in_specs=in_specs,
        out_specs=[pl.BlockSpec((NODE_BLK, HID), lambda i: (i, 0)),
                   pl.BlockSpec((NODE_BLK, TD_W), lambda i: (i, 0)),
                   pl.BlockSpec((NODE_BLK, TS_W), lambda i: (i, 0))],
        out_shape=[jax.ShapeDtypeStruct((N, HID), jnp.float32),
                   jax.ShapeDtypeStruct((N, TD_W), jnp.float32),
                   jax.ShapeDtypeStruct((N, TS_W), jnp.float32)],
        interpret=_INTERPRET,
    )(*args)


def _tail_body(acc_ref, x_ref, se0_ref, se1_ref, se2_ref, wft_ref, wdt_ref,
               xo_ref):
    acc = acc_ref[...]
    x = x_ref[...]
    se = (se0_ref[...], se1_ref[...], se2_ref[...])
    den = acc[0, :, 48:64] + acc[1, :, 48:64]
    wft = wft_ref[...]
    wdt = wdt_ref[...]
    outs, Ps, Ds = [], [], []
    for c in range(3):
        num = acc[0, :, c * 16:(c + 1) * 16] + acc[1, :, c * 16:(c + 1) * 16]
        oc = num / (den + 1e-16)
        outs.append(oc)
        tmp = jnp.concatenate([x[:, c:c + 1], oc, se[c]], axis=1)
        Ps.append(tmp @ wft)
        Ds.append(tmp @ wdt)
    dot = Ps[0] * Ds[0] + Ps[1] * Ds[1] + Ps[2] * Ds[2]
    dsq = Ds[0] * Ds[0] + Ds[1] * Ds[1] + Ds[2] * Ds[2]
    coef = dot / (dsq + 1e-6)
    mask = dot >= 0.0
    deltas = []
    for c in range(3):
        neg = jnp.where(mask, Ps[c], Ps[c] - coef * Ds[c])
        res = 0.2 * Ps[c] + 0.8 * neg
        delta = jnp.mean(outs[c], axis=-1, keepdims=True) \
            + jnp.mean(res, axis=-1, keepdims=True)
        deltas.append(x[:, c:c + 1] + delta)
    xo_ref[...] = jnp.concatenate(deltas, axis=1)


def _tail(acc2, x, se0, se1, se2, wft, wdt):
    args = [acc2, x, se0, se1, se2, wft, wdt]
    in_specs = [pl.BlockSpec((2, NODE_BLK, 64), lambda i: (0, i, 0)),
                pl.BlockSpec((NODE_BLK, 3), lambda i: (i, 0)),
                pl.BlockSpec((NODE_BLK, 16), lambda i: (i, 0)),
                pl.BlockSpec((NODE_BLK, 16), lambda i: (i, 0)),
                pl.BlockSpec((NODE_BLK, 16), lambda i: (i, 0)),
                _full_spec(wft), _full_spec(wdt)]
    return pl.pallas_call(
        _tail_body,
        grid=(N // NODE_BLK,),
        in_specs=in_specs,
        out_specs=pl.BlockSpec((NODE_BLK, 3), lambda i: (i, 0)),
        out_shape=jax.ShapeDtypeStruct((N, 3), jnp.float32),
        interpret=_INTERPRET,
    )(*args)


CHUNK = 128
NROWS = E // CHUNK
NWORK = 32
ROWS_PER_W = -(-NROWS // NWORK)
NODES_PER_SUB = N // 16

_SC_MESH = plsc.VectorSubcoreMesh(core_axis_name="c", subcore_axis_name="s")


def _gather(td, ts, dst_m, src_m):
    @functools.partial(
        pl.kernel,
        out_type=[jax.ShapeDtypeStruct((E, TD_W), jnp.float32),
                  jax.ShapeDtypeStruct((E, TS_W), jnp.float32)],
        mesh=_SC_MESH,
        scratch_types=[pltpu.VMEM((CHUNK,), jnp.int32),
                       pltpu.VMEM((CHUNK,), jnp.int32),
                       pltpu.VMEM((CHUNK, TD_W), jnp.float32),
                       pltpu.VMEM((CHUNK, TS_W), jnp.float32),
                       pltpu.SemaphoreType.DMA,
                       pltpu.SemaphoreType.DMA],
    )
    def gk(td_hbm, ts_hbm, dm_hbm, sm_hbm, gd_hbm, gs_hbm,
           idx_d, idx_s, rows_d, rows_s, sem_d, sem_s):
        wid = lax.axis_index("s") * 2 + lax.axis_index("c")

        @pl.loop(0, ROWS_PER_W)
        def _(i):
            r = wid + i * NWORK

            @pl.when(r < NROWS)
            def _():
                pltpu.sync_copy(dm_hbm.at[r], idx_d)
                pltpu.sync_copy(sm_hbm.at[r], idx_s)
                cd = pltpu.async_copy(td_hbm.at[idx_d], rows_d, sem_d)
                cs = pltpu.async_copy(ts_hbm.at[idx_s], rows_s, sem_s)
                cd.wait()
                cs.wait()
                pltpu.sync_copy(rows_d, gd_hbm.at[pl.ds(r * CHUNK, CHUNK)])
                pltpu.sync_copy(rows_s, gs_hbm.at[pl.ds(r * CHUNK, CHUNK)])

    return gk(td, ts, dst_m, src_m)


def _gather_x(xpad, dst_m, src_m):
    @functools.partial(
        pl.kernel,
        out_type=[jax.ShapeDtypeStruct((E, 16), jnp.float32),
                  jax.ShapeDtypeStruct((E, 16), jnp.float32)],
        mesh=_SC_MESH,
        scratch_types=[pltpu.VMEM((CHUNK,), jnp.int32),
                       pltpu.VMEM((CHUNK,), jnp.int32),
                       pltpu.VMEM((CHUNK, 16), jnp.float32),
                       pltpu.VMEM((CHUNK, 16), jnp.float32),
                       pltpu.SemaphoreType.DMA,
                       pltpu.SemaphoreType.DMA],
        compiler_params=pltpu.CompilerParams(use_tc_tiling_on_sc=False),
    )
    def gxk(x_hbm, dm_hbm, sm_hbm, gxd_hbm, gxs_hbm,
            idx_d, idx_s, rows_d, rows_s, sem_d, sem_s):
        wid = lax.axis_index("s") * 2 + lax.axis_index("c")

        @pl.loop(0, ROWS_PER_W)
        def _(i):
            r = wid + i * NWORK

            @pl.when(r < NROWS)
            def _():
                pltpu.sync_copy(dm_hbm.at[r], idx_d)
                pltpu.sync_copy(sm_hbm.at[r], idx_s)
                cd = pltpu.async_copy(x_hbm.at[idx_d], rows_d, sem_d)
                cs = pltpu.async_copy(x_hbm.at[idx_s], rows_s, sem_s)
                cd.wait()
                cs.wait()
                pltpu.sync_copy(rows_d, gxd_hbm.at[pl.ds(r * CHUNK, CHUNK)])
                pltpu.sync_copy(rows_s, gxs_hbm.at[pl.ds(r * CHUNK, CHUNK)])

    return gxk(xpad, dst_m, src_m)


def _scatter_add(rows, dst_m, width):
    zeros = jnp.zeros((NODES_PER_SUB, width), jnp.float32)

    @functools.partial(
        pl.kernel,
        out_type=jax.ShapeDtypeStruct((2, N, width), jnp.float32),
        mesh=_SC_MESH,
        scratch_types=[pltpu.VMEM((CHUNK,), jnp.int32),
                       pltpu.VMEM((CHUNK, width), jnp.float32),
                       pltpu.VMEM_SHARED((N, width), jnp.float32)],
        compiler_params=pltpu.CompilerParams(use_tc_tiling_on_sc=False),
    )
    def sk(rows_hbm, dm_hbm, z_hbm, acc_hbm, idx, buf, shared):
        cid = lax.axis_index("c")
        sid = lax.axis_index("s")
        wid = sid * 2 + cid
        pltpu.sync_copy(z_hbm, shared.at[pl.ds(sid * NODES_PER_SUB,
                                               NODES_PER_SUB)])
        plsc.subcore_barrier()

        @pl.loop(0, ROWS_PER_W)
        def _(i):
            r = wid + i * NWORK

            @pl.when(r < NROWS)
            def _():
                pltpu.sync_copy(dm_hbm.at[r], idx)
                pltpu.sync_copy(rows_hbm.at[pl.ds(r * CHUNK, CHUNK)], buf)
                pltpu.sync_copy(buf, shared.at[idx], add=True)

        plsc.subcore_barrier()
        pltpu.sync_copy(
            shared.at[pl.ds(sid * NODES_PER_SUB, NODES_PER_SUB)],
            acc_hbm.at[cid, pl.ds(sid * NODES_PER_SUB, NODES_PER_SUB)])

    return sk(rows, dst_m, zeros)


def _prep_kv_mlp(p):
    w1 = p["W1"]
    return {
        "ea": w1[0:EDGE_DIM],
        "rf": w1[EDGE_DIM:EDGE_DIM + NG * EDGE_DIM],
        "hd": w1[84:84 + HID],
        "hs": w1[84 + HID:84 + 2 * HID],
        "iv": w1[84 + 2 * HID:],
        "b1": p["b1"].reshape(1, -1),
        "g": p["g"].reshape(1, -1),
        "be": p["be"].reshape(1, -1),
        "W2": p["W2"],
        "b2": p["b2"].reshape(1, -1),
    }


def _prep_q_mlp(p):
    return (p["W1"], p["b1"].reshape(1, -1), p["g"].reshape(1, -1),
            p["be"].reshape(1, -1), p["W2"], p["b2"].reshape(1, -1))


def kernel(h, x, edge_attr, edge_index, invar_ligand_shape, ligand_shape_emb,
           topo_out, e_w, params):
    del topo_out
    src = edge_index[0]
    dst = edge_index[1]
    dst_m = dst.reshape(NROWS, CHUNK)
    src_m = src.reshape(NROWS, CHUNK)
    ew = e_w.reshape(E, 1)
    xpad = jnp.pad(x, ((0, 0), (0, 13)))

    perm = np.array([(j % HEADS) * DH + j // HEADS for j in range(HID)],
                    dtype=np.int32)

    px = params["x2h"]
    hk = _prep_kv_mlp(px["hk"])
    hv = _prep_kv_mlp(px["hv"])
    hq = _prep_q_mlp(px["hq"])
    no = px["node_out"]
    n_w1 = jnp.concatenate([no["W1"][0:HID][perm], no["W1"][HID:]], axis=0)
    wn = (n_w1, no["b1"].reshape(1, -1), no["g"].reshape(1, -1),
          no["be"].reshape(1, -1), no["W2"], no["b2"].reshape(1, -1))

    ph = params["h2x"]
    xk = _prep_kv_mlp(ph["xk"])
    xv = _prep_kv_mlp(ph["xv"])
    xq = _prep_q_mlp(ph["xq"])
    wft = ph["Wf"].T
    wdt = ph["Wd"].T

    def kv_pack(m):
        return (m["hd"], m["iv"], m["b1"], m["hs"])

    gxd, gxs = _gather_x(xpad, dst_m, src_m)

    td1, ts1 = _node_tables(h, invar_ligand_shape,
                            kv_pack(hk), kv_pack(hv), hq)
    gd1, gs1 = _gather(td1, ts1, dst_m, src_m)
    hv_w2p = hv["W2"][:, perm]
    hv_b2p = hv["b2"][:, perm]
    s1 = _edge1(gd1, gs1, gxd, gxs, edge_attr, ew,
                (hk["ea"], hk["rf"], hk["g"], hk["be"], hk["W2"], hk["b2"]),
                (hv["ea"], hv["rf"], hv["g"], hv["be"], hv_w2p, hv_b2p))
    acc1 = _scatter_add(s1, dst_m, HID + HEADS)

    h_out, td2, ts2 = _node2(acc1, h, invar_ligand_shape, wn,
                             kv_pack(xk), kv_pack(xv), xq)

    gd2, gs2 = _gather(td2, ts2, dst_m, src_m)
    s2 = _edge2(gd2, gs2, gxd, gxs, edge_attr, ew,
                (xk["ea"], xk["rf"], xk["g"], xk["be"], xk["W2"], xk["b2"]),
                (xv["ea"], xv["rf"], xv["g"], xv["be"], xv["W2"],
                 xv["b2"]))
    acc2 = _scatter_add(s2, dst_m, 64)

    se0 = ligand_shape_emb[:, :, 0]
    se1 = ligand_shape_emb[:, :, 1]
    se2 = ligand_shape_emb[:, :, 2]
    x_out = _tail(acc2, x, se0, se1, se2, wft, wdt)
    return h_out, x_out
